# Initial kernel scaffold; baseline (speedup 1.0000x reference)
import jax
import jax.numpy as jnp
from jax.experimental import pallas as pl


def kernel(hidden_states, cos, sin, attention_mask, Wq, Wk, Wv, Wo):
    raise NotImplementedError("write your pallas kernel here")


# speedup vs baseline: 1.1544x; 1.1544x over previous
import functools

import jax
import jax.numpy as jnp
from jax.experimental import pallas as pl

S, D = 2048, 2048
HQ, HKV, HD = 16, 4, 128
N_REP = HQ // HKV
NH = HQ + 2 * HKV
SCALING = HD ** -0.5
QB = 512
MB = 256


def _qkv_rope_kernel(x_ref, w_ref, cos_ref, sin_ref, out_ref):
    h = pl.program_id(0)
    y = jax.lax.dot_general(
        x_ref[...], w_ref[0],
        (((1,), (1,)), ((), ())),
        preferred_element_type=jnp.float32,
    )
    half = HD // 2
    rot = jnp.concatenate([-y[:, half:], y[:, :half]], axis=-1)
    roped = y * cos_ref[...] + rot * sin_ref[...]
    out_ref[0] = jnp.where(h < HQ + HKV, roped, y)


def _attn_kernel(q_ref, k_ref, v_ref, out_ref):
    i = pl.program_id(1)
    q = q_ref[0] * SCALING
    s = jax.lax.dot_general(
        q, k_ref[0],
        (((1,), (1,)), ((), ())),
        preferred_element_type=jnp.float32,
    )
    rows = i * QB + jax.lax.broadcasted_iota(jnp.int32, (QB, S), 0)
    cols = jax.lax.broadcasted_iota(jnp.int32, (QB, S), 1)
    s = jnp.where(cols <= rows, s, -jnp.inf)
    m = jnp.max(s, axis=-1, keepdims=True)
    p = jnp.exp(s - m)
    p = p / jnp.sum(p, axis=-1, keepdims=True)
    out_ref[...] = jax.lax.dot_general(
        p, v_ref[0],
        (((1,), (0,)), ((), ())),
        preferred_element_type=jnp.float32,
    )


def _out_proj_kernel(x_ref, w_ref, out_ref):
    out_ref[...] = jax.lax.dot_general(
        x_ref[...], w_ref[...],
        (((1,), (1,)), ((), ())),
        preferred_element_type=jnp.float32,
    )


@jax.jit
def _run(x, cs, sn, w_all, Wo):
    qkv = pl.pallas_call(
        _qkv_rope_kernel,
        grid=(NH,),
        in_specs=[
            pl.BlockSpec((S, D), lambda h: (0, 0)),
            pl.BlockSpec((1, HD, D), lambda h: (h, 0, 0)),
            pl.BlockSpec((S, HD), lambda h: (0, 0)),
            pl.BlockSpec((S, HD), lambda h: (0, 0)),
        ],
        out_specs=pl.BlockSpec((1, S, HD), lambda h: (h, 0, 0)),
        out_shape=jax.ShapeDtypeStruct((NH, S, HD), jnp.float32),
    )(x, w_all, cs, sn)

    attn = pl.pallas_call(
        _attn_kernel,
        grid=(HQ, S // QB),
        in_specs=[
            pl.BlockSpec((1, QB, HD), lambda h, i: (h, i, 0)),
            pl.BlockSpec((1, S, HD), lambda h, i: (HQ + h // N_REP, 0, 0)),
            pl.BlockSpec((1, S, HD), lambda h, i: (HQ + HKV + h // N_REP, 0, 0)),
        ],
        out_specs=pl.BlockSpec((QB, HD), lambda h, i: (i, h)),
        out_shape=jax.ShapeDtypeStruct((S, HQ * HD), jnp.float32),
    )(qkv, qkv, qkv)

    out = pl.pallas_call(
        _out_proj_kernel,
        grid=(S // MB,),
        in_specs=[
            pl.BlockSpec((MB, HQ * HD), lambda i: (i, 0)),
            pl.BlockSpec((D, HQ * HD), lambda i: (0, 0)),
        ],
        out_specs=pl.BlockSpec((MB, D), lambda i: (i, 0)),
        out_shape=jax.ShapeDtypeStruct((S, D), jnp.float32),
    )(attn, Wo)
    return out


def kernel(hidden_states, cos, sin, attention_mask, Wq, Wk, Wv, Wo):
    b = hidden_states.shape[0]
    x = hidden_states[0]
    w_all = jnp.concatenate(
        [Wq.reshape(HQ, HD, D), Wk.reshape(HKV, HD, D), Wv.reshape(HKV, HD, D)],
        axis=0,
    )
    out = _run(x, cos[0], sin[0], w_all, Wo)
    return out.reshape(b, S, D)


# speedup vs baseline: 1.5081x; 1.3064x over previous
import functools

import jax
import jax.numpy as jnp
from jax.experimental import pallas as pl

S, D = 2048, 2048
HQ, HKV, HD = 16, 4, 128
N_REP = HQ // HKV
NH = HQ + 2 * HKV
SCALING = HD ** -0.5
QB = 512
MB = 256


def _qkv_rope_kernel(x_ref, w_ref, cos_ref, sin_ref, out_ref):
    h = pl.program_id(0)
    y = jax.lax.dot_general(
        x_ref[...], w_ref[0],
        (((1,), (1,)), ((), ())),
        preferred_element_type=jnp.float32,
    )
    half = HD // 2
    rot = jnp.concatenate([-y[:, half:], y[:, :half]], axis=-1)
    roped = y * cos_ref[...] + rot * sin_ref[...]
    out_ref[0] = jnp.where(h < HQ + HKV, roped, y)


def _attn_kernel(q_ref, k_ref, v_ref, out_ref):
    i = pl.program_id(1)
    q = q_ref[0] * SCALING

    rows = jax.lax.broadcasted_iota(jnp.int32, (QB, QB), 0)
    cols = jax.lax.broadcasted_iota(jnp.int32, (QB, QB), 1)
    diag_mask = cols <= rows

    def body(j, carry):
        acc, m, l = carry
        k_j = k_ref[0, pl.ds(j * QB, QB), :]
        v_j = v_ref[0, pl.ds(j * QB, QB), :]
        s = jax.lax.dot_general(
            q, k_j,
            (((1,), (1,)), ((), ())),
            preferred_element_type=jnp.float32,
        )
        s = jnp.where(jnp.logical_or(j < i, diag_mask), s, -jnp.inf)
        m_new = jnp.maximum(m, jnp.max(s, axis=-1, keepdims=True))
        p = jnp.exp(s - m_new)
        corr = jnp.exp(m - m_new)
        l = l * corr + jnp.sum(p, axis=-1, keepdims=True)
        acc = acc * corr + jax.lax.dot_general(
            p, v_j,
            (((1,), (0,)), ((), ())),
            preferred_element_type=jnp.float32,
        )
        return acc, m_new, l

    acc = jnp.zeros((QB, HD), jnp.float32)
    m0 = jnp.full((QB, 1), -jnp.inf, jnp.float32)
    l0 = jnp.zeros((QB, 1), jnp.float32)
    acc, m, l = jax.lax.fori_loop(0, i + 1, body, (acc, m0, l0))
    out_ref[...] = acc / l


def _out_proj_kernel(x_ref, w_ref, out_ref):
    out_ref[...] = jax.lax.dot_general(
        x_ref[...], w_ref[...],
        (((1,), (1,)), ((), ())),
        preferred_element_type=jnp.float32,
    )


@jax.jit
def _run(x, cs, sn, w_all, Wo):
    qkv = pl.pallas_call(
        _qkv_rope_kernel,
        grid=(NH,),
        in_specs=[
            pl.BlockSpec((S, D), lambda h: (0, 0)),
            pl.BlockSpec((1, HD, D), lambda h: (h, 0, 0)),
            pl.BlockSpec((S, HD), lambda h: (0, 0)),
            pl.BlockSpec((S, HD), lambda h: (0, 0)),
        ],
        out_specs=pl.BlockSpec((1, S, HD), lambda h: (h, 0, 0)),
        out_shape=jax.ShapeDtypeStruct((NH, S, HD), jnp.float32),
    )(x, w_all, cs, sn)

    attn = pl.pallas_call(
        _attn_kernel,
        grid=(HQ, S // QB),
        in_specs=[
            pl.BlockSpec((1, QB, HD), lambda h, i: (h, i, 0)),
            pl.BlockSpec((1, S, HD), lambda h, i: (HQ + h // N_REP, 0, 0)),
            pl.BlockSpec((1, S, HD), lambda h, i: (HQ + HKV + h // N_REP, 0, 0)),
        ],
        out_specs=pl.BlockSpec((QB, HD), lambda h, i: (i, h)),
        out_shape=jax.ShapeDtypeStruct((S, HQ * HD), jnp.float32),
    )(qkv, qkv, qkv)

    out = pl.pallas_call(
        _out_proj_kernel,
        grid=(S // MB,),
        in_specs=[
            pl.BlockSpec((MB, HQ * HD), lambda i: (i, 0)),
            pl.BlockSpec((D, HQ * HD), lambda i: (0, 0)),
        ],
        out_specs=pl.BlockSpec((MB, D), lambda i: (i, 0)),
        out_shape=jax.ShapeDtypeStruct((S, D), jnp.float32),
    )(attn, Wo)
    return out


def kernel(hidden_states, cos, sin, attention_mask, Wq, Wk, Wv, Wo):
    b = hidden_states.shape[0]
    x = hidden_states[0]
    w_all = jnp.concatenate(
        [Wq.reshape(HQ, HD, D), Wk.reshape(HKV, HD, D), Wv.reshape(HKV, HD, D)],
        axis=0,
    )
    out = _run(x, cos[0], sin[0], w_all, Wo)
    return out.reshape(b, S, D)
